# baseline (speedup 1.0000x reference)
import jax
import jax.numpy as jnp
from jax.experimental import pallas as pl
from jax.experimental.pallas import tpu as pltpu

BN = 1024


def _body(x_ref, e_ref, dist_ref, ind_ref, q_ref, idx_s):
    i = pl.program_id(0)
    g = pl.num_programs(0) - 1
    e = e_ref[...]
    c = e.shape[0]

    @pl.when(i > 0)
    def _quant_prev():
        idxp = idx_s[...]
        iota = jax.lax.broadcasted_iota(
            jnp.int32, (idxp.shape[0], c), 1).astype(jnp.float32)
        oh = (iota == idxp).astype(jnp.float32)
        q_ref[...] = jax.lax.dot_general(oh, e, (((1,), (0,)), ((), ())),
                                         preferred_element_type=jnp.float32)

    @pl.when(i < g)
    def _dist_cur():
        xb = x_ref[...]
        d = jax.lax.dot_general(xb, e, (((1,), (1,)), ((), ())),
                                preferred_element_type=jnp.float32)
        dist_ref[...] = d
        m = jnp.max(d, axis=-1, keepdims=True)
        iota = jax.lax.broadcasted_iota(
            jnp.int32, d.shape, 1).astype(jnp.float32)
        w = jnp.where(d == m, iota, float(c))
        idx = jnp.min(w, axis=-1, keepdims=True)
        ind_ref[0, 0, :] = jnp.transpose(idx.astype(jnp.int32), (1, 0))[0]
        idx_s[...] = idx


def kernel(x, embed):
    x = x.astype(jnp.float32)
    b, n, d = x.shape
    h, c, _ = embed.shape
    N = b * n
    xf = x.reshape(N, d)
    ef = embed.reshape(c, d)
    g = N // BN
    last = g - 1
    dist, ind3, quant = pl.pallas_call(
        _body,
        grid=(g + 1,),
        in_specs=[
            pl.BlockSpec((BN, d), lambda i: (jnp.minimum(i, last), 0)),
            pl.BlockSpec((c, d), lambda i: (0, 0)),
        ],
        out_specs=[
            pl.BlockSpec((BN, c), lambda i: (jnp.minimum(i, last), 0)),
            pl.BlockSpec((1, 1, BN), lambda i: (jnp.minimum(i, last), 0, 0)),
            pl.BlockSpec((BN, d), lambda i: (jnp.maximum(i, 1) - 1, 0)),
        ],
        out_shape=[
            jax.ShapeDtypeStruct((N, c), jnp.float32),
            jax.ShapeDtypeStruct((g, 1, BN), jnp.int32),
            jax.ShapeDtypeStruct((N, d), jnp.float32),
        ],
        scratch_shapes=[pltpu.VMEM((BN, 1), jnp.float32)],
    )(xf, ef)
    quantize = quant.reshape(b, n, d)
    embed_ind = ind3.reshape(b, n)
    dist_out = dist.reshape(h, b, n, c)
    return quantize, embed_ind, dist_out


# speedup vs baseline: 1.1102x; 1.1102x over previous
import jax
import jax.numpy as jnp
from jax.experimental import pallas as pl
from jax.experimental.pallas import tpu as pltpu

BN = 2048


def _body(x_ref, e_ref, dist_ref, ind_ref, q_ref, idx_s):
    i = pl.program_id(0)
    g = pl.num_programs(0) - 1
    e = e_ref[...]
    c = e.shape[0]

    @pl.when(i > 0)
    def _quant_prev():
        idxp = idx_s[...]
        iota = jax.lax.broadcasted_iota(
            jnp.int32, (idxp.shape[0], c), 1).astype(jnp.float32)
        oh = (iota == idxp).astype(jnp.float32)
        q_ref[...] = jax.lax.dot_general(oh, e, (((1,), (0,)), ((), ())),
                                         preferred_element_type=jnp.float32)

    @pl.when(i < g)
    def _dist_cur():
        xb = x_ref[...]
        d = jax.lax.dot_general(xb, e, (((1,), (1,)), ((), ())),
                                preferred_element_type=jnp.float32)
        dist_ref[...] = d
        m = jnp.max(d, axis=-1, keepdims=True)
        iota = jax.lax.broadcasted_iota(
            jnp.int32, d.shape, 1).astype(jnp.float32)
        w = jnp.where(d == m, iota, float(c))
        idx = jnp.min(w, axis=-1, keepdims=True)
        ind_ref[0, 0, :] = jnp.transpose(idx.astype(jnp.int32), (1, 0))[0]
        idx_s[...] = idx


def kernel(x, embed):
    x = x.astype(jnp.float32)
    b, n, d = x.shape
    h, c, _ = embed.shape
    N = b * n
    xf = x.reshape(N, d)
    ef = embed.reshape(c, d)
    g = N // BN
    last = g - 1
    dist, ind3, quant = pl.pallas_call(
        _body,
        grid=(g + 1,),
        in_specs=[
            pl.BlockSpec((BN, d), lambda i: (jnp.minimum(i, last), 0)),
            pl.BlockSpec((c, d), lambda i: (0, 0)),
        ],
        out_specs=[
            pl.BlockSpec((BN, c), lambda i: (jnp.minimum(i, last), 0)),
            pl.BlockSpec((1, 1, BN), lambda i: (jnp.minimum(i, last), 0, 0)),
            pl.BlockSpec((BN, d), lambda i: (jnp.maximum(i, 1) - 1, 0)),
        ],
        out_shape=[
            jax.ShapeDtypeStruct((N, c), jnp.float32),
            jax.ShapeDtypeStruct((g, 1, BN), jnp.int32),
            jax.ShapeDtypeStruct((N, d), jnp.float32),
        ],
        scratch_shapes=[pltpu.VMEM((BN, 1), jnp.float32)],
    )(xf, ef)
    quantize = quant.reshape(b, n, d)
    embed_ind = ind3.reshape(b, n)
    dist_out = dist.reshape(h, b, n, c)
    return quantize, embed_ind, dist_out
